# Initial kernel scaffold; baseline (speedup 1.0000x reference)
import jax
import jax.numpy as jnp
from jax.experimental import pallas as pl


def kernel(x_joint, x_grain, edge_attr_jj, params, edge_index_jj, edge_index_jg, edge_index_gj):
    raise NotImplementedError("write your pallas kernel here")


# speedup vs baseline: 8.9204x; 8.9204x over previous
import functools

import jax
import jax.numpy as jnp
from jax import lax
from jax.experimental import pallas as pl
from jax.experimental.pallas import tpu as pltpu
from jax.experimental.pallas import tpu_sc as plsc

NJ, NG = 50000, 25000
OUT = 16
D = 16
EJJ, EJG, EGJ = 800000, 400000, 400000

NC, NS = 2, 16
NW = NC * NS
MB = 128

NJP = 50176
NGP = 25088
EJJP = 819200
EJGP = 425984
EGJP = 425984

GATES = ('i', 'f', 'g', 'o')


def _make_segsum(epad, ndp):
    rt = epad // MB // NW
    zb = ndp // NS

    mesh = plsc.VectorSubcoreMesh(core_axis_name="c", subcore_axis_name="s",
                                  num_cores=NC, num_subcores=NS)

    @functools.partial(
        pl.kernel, mesh=mesh,
        out_type=jax.ShapeDtypeStruct((NC, ndp, D), jnp.float32),
        compiler_params=pltpu.CompilerParams(use_tc_tiling_on_sc=False),
        scratch_types=[
            pltpu.VMEM((rt, MB), jnp.int32),
            pltpu.VMEM((rt, MB), jnp.int32),
            pltpu.VMEM((MB, D), jnp.float32),
            pltpu.VMEM_SHARED((ndp, D), jnp.float32),
            pltpu.SemaphoreType.DMA,
        ])
    def seg(x_hbm, src_hbm, dst_hbm, zeros_hbm, out_hbm,
            sidx, didx, rows, acc, sem):
        cid = lax.axis_index("c")
        sid = lax.axis_index("s")
        wid = cid * NS + sid

        pltpu.sync_copy(zeros_hbm.at[pl.ds(sid * zb, zb)],
                        acc.at[pl.ds(sid * zb, zb)])
        pltpu.sync_copy(src_hbm.at[pl.ds(wid * rt, rt)], sidx)
        pltpu.sync_copy(dst_hbm.at[pl.ds(wid * rt, rt)], didx)
        plsc.subcore_barrier()

        def mb_body(j, carry):
            pltpu.async_copy(x_hbm.at[sidx.at[j]], rows, sem).wait()
            pltpu.sync_copy(rows, acc.at[didx.at[j]], add=True)
            return carry

        lax.fori_loop(0, rt, mb_body, 0)
        plsc.subcore_barrier()

        pltpu.sync_copy(acc.at[pl.ds(sid * zb, zb)],
                        out_hbm.at[cid, pl.ds(sid * zb, zb)])

    return seg


def _make_pairgather():
    rt = EJJP // MB // NW
    mesh = plsc.VectorSubcoreMesh(core_axis_name="c", subcore_axis_name="s",
                                  num_cores=NC, num_subcores=NS)

    @functools.partial(
        pl.kernel, mesh=mesh,
        out_type=(jax.ShapeDtypeStruct((EJJP, D), jnp.float32),
                  jax.ShapeDtypeStruct((EJJP, D), jnp.float32)),
        compiler_params=pltpu.CompilerParams(use_tc_tiling_on_sc=False),
        scratch_types=[
            pltpu.VMEM((rt, MB), jnp.int32),
            pltpu.VMEM((rt, MB), jnp.int32),
            pltpu.VMEM((MB, D), jnp.float32),
            pltpu.VMEM((MB, D), jnp.float32),
            pltpu.SemaphoreType.DMA,
            pltpu.SemaphoreType.DMA,
        ])
    def pair(ta_hbm, tb_hbm, src_hbm, dst_hbm, gs_hbm, gd_hbm,
             sidx, didx, rs, rd, sem_s, sem_d):
        cid = lax.axis_index("c")
        sid = lax.axis_index("s")
        wid = cid * NS + sid
        pltpu.sync_copy(src_hbm.at[pl.ds(wid * rt, rt)], sidx)
        pltpu.sync_copy(dst_hbm.at[pl.ds(wid * rt, rt)], didx)

        def mb_body(j, carry):
            cs = pltpu.async_copy(ta_hbm.at[sidx.at[j]], rs, sem_s)
            cd = pltpu.async_copy(tb_hbm.at[didx.at[j]], rd, sem_d)
            cs.wait()
            cd.wait()
            base = (wid * rt + j) * MB
            pltpu.sync_copy(rs, gs_hbm.at[pl.ds(base, MB)])
            pltpu.sync_copy(rd, gd_hbm.at[pl.ds(base, MB)])
            return carry

        lax.fori_loop(0, rt, mb_body, 0)

    return pair


_BN = 512


def _cell_body_j(sjj, sgj, cjj, cgj, x, h, c, w, b, h2, c2):
    a1 = (sjj[0] + sjj[1]) / jnp.clip(cjj[0] + cjj[1], 1.0, None)
    a2 = (sgj[0] + sgj[1]) / jnp.clip(cgj[0] + cgj[1], 1.0, None)
    wv = w[...]
    z = (jnp.dot(a1, wv[0:16, :], preferred_element_type=jnp.float32,
             precision=lax.Precision.HIGHEST)
         + jnp.dot(a2, wv[16:32, :], preferred_element_type=jnp.float32,
             precision=lax.Precision.HIGHEST)
         + jnp.dot(x[...], wv[32:48, :], preferred_element_type=jnp.float32,
             precision=lax.Precision.HIGHEST)
         + jnp.dot(h[...], wv[48:64, :], preferred_element_type=jnp.float32,
             precision=lax.Precision.HIGHEST)
         + b[...])
    cn = (jax.nn.sigmoid(z[:, 16:32]) * c[...]
          + jax.nn.sigmoid(z[:, 0:16]) * jnp.tanh(z[:, 32:48]))
    h2[...] = jax.nn.sigmoid(z[:, 48:64]) * jnp.tanh(cn)
    c2[...] = cn


def _cell_body_g(sjg, cjg, x, h, c, w, b, h2, c2):
    a1 = (sjg[0] + sjg[1]) / jnp.clip(cjg[0] + cjg[1], 1.0, None)
    wv = w[...]
    z = (jnp.dot(a1, wv[0:16, :], preferred_element_type=jnp.float32,
             precision=lax.Precision.HIGHEST)
         + jnp.dot(x[...], wv[16:32, :], preferred_element_type=jnp.float32,
             precision=lax.Precision.HIGHEST)
         + jnp.dot(h[...], wv[32:48, :], preferred_element_type=jnp.float32,
             precision=lax.Precision.HIGHEST)
         + b[...])
    cn = (jax.nn.sigmoid(z[:, 16:32]) * c[...]
          + jax.nn.sigmoid(z[:, 0:16]) * jnp.tanh(z[:, 32:48]))
    h2[...] = jax.nn.sigmoid(z[:, 48:64]) * jnp.tanh(cn)
    c2[...] = cn


def _make_cell_j(ndp):
    grid = (ndp // _BN,)
    part = lambda i: (0, i, 0)
    row = lambda i: (i, 0)
    return pl.pallas_call(
        _cell_body_j,
        grid=grid,
        in_specs=[
            pl.BlockSpec((NC, _BN, D), part),
            pl.BlockSpec((NC, _BN, D), part),
            pl.BlockSpec((NC, _BN, D), part),
            pl.BlockSpec((NC, _BN, D), part),
            pl.BlockSpec((_BN, D), row),
            pl.BlockSpec((_BN, D), row),
            pl.BlockSpec((_BN, D), row),
            pl.BlockSpec((64, 64), lambda i: (0, 0)),
            pl.BlockSpec((1, 64), lambda i: (0, 0)),
        ],
        out_specs=[pl.BlockSpec((_BN, D), row), pl.BlockSpec((_BN, D), row)],
        out_shape=[jax.ShapeDtypeStruct((ndp, D), jnp.float32),
                   jax.ShapeDtypeStruct((ndp, D), jnp.float32)],
    )


def _make_cell_g(ndp):
    grid = (ndp // _BN,)
    part = lambda i: (0, i, 0)
    row = lambda i: (i, 0)
    return pl.pallas_call(
        _cell_body_g,
        grid=grid,
        in_specs=[
            pl.BlockSpec((NC, _BN, D), part),
            pl.BlockSpec((NC, _BN, D), part),
            pl.BlockSpec((_BN, D), row),
            pl.BlockSpec((_BN, D), row),
            pl.BlockSpec((_BN, D), row),
            pl.BlockSpec((48, 64), lambda i: (0, 0)),
            pl.BlockSpec((1, 64), lambda i: (0, 0)),
        ],
        out_specs=[pl.BlockSpec((_BN, D), row), pl.BlockSpec((_BN, D), row)],
        out_shape=[jax.ShapeDtypeStruct((ndp, D), jnp.float32),
                   jax.ShapeDtypeStruct((ndp, D), jnp.float32)],
    )


def _project_body(h, wa, wb, ta, tb):
    hv = h[...]
    ta[...] = jnp.dot(hv, wa[...], preferred_element_type=jnp.float32,
                      precision=lax.Precision.HIGHEST)
    tb[...] = jnp.dot(hv, wb[...], preferred_element_type=jnp.float32,
                      precision=lax.Precision.HIGHEST)


def _make_project():
    grid = (NJP // _BN,)
    return pl.pallas_call(
        _project_body,
        grid=grid,
        in_specs=[pl.BlockSpec((_BN, D), lambda i: (i, 0)),
                  pl.BlockSpec((D, D), lambda i: (0, 0)),
                  pl.BlockSpec((D, D), lambda i: (0, 0))],
        out_specs=[pl.BlockSpec((_BN, D), lambda i: (i, 0)),
                   pl.BlockSpec((_BN, D), lambda i: (i, 0))],
        out_shape=[jax.ShapeDtypeStruct((NJP, D), jnp.float32),
                   jax.ShapeDtypeStruct((NJP, D), jnp.float32)],
    )


_BE = 4096


def _final_body(gs, gd, ea, wt, b, out):
    z = (gs[:, 0:2] + gd[:, 0:2]
         + jnp.dot(ea[...], wt[...], preferred_element_type=jnp.float32,
                   precision=lax.Precision.HIGHEST)
         + b[...])
    out[...] = jax.nn.sigmoid(z)


def _make_final():
    grid = (EJJP // _BE,)
    return pl.pallas_call(
        _final_body,
        grid=grid,
        in_specs=[pl.BlockSpec((_BE, D), lambda i: (i, 0)),
                  pl.BlockSpec((_BE, D), lambda i: (i, 0)),
                  pl.BlockSpec((_BE, 2), lambda i: (i, 0)),
                  pl.BlockSpec((2, 2), lambda i: (0, 0)),
                  pl.BlockSpec((1, 2), lambda i: (0, 0))],
        out_specs=pl.BlockSpec((_BE, 2), lambda i: (i, 0)),
        out_shape=jax.ShapeDtypeStruct((EJJP, 2), jnp.float32),
    )


def _pad_rows(x, n, d=D):
    return jnp.zeros((n, d), jnp.float32).at[:x.shape[0], :x.shape[1]].set(x)


def _pad_edges(ei, epad, dst_pad):
    e = ei.shape[1]
    src = jnp.full((epad,), 0, jnp.int32).at[:e].set(ei[0].astype(jnp.int32))
    dst = jnp.full((epad,), dst_pad, jnp.int32).at[:e].set(
        ei[1].astype(jnp.int32))
    return src.reshape(epad // MB, MB), dst.reshape(epad // MB, MB)


def _pad16(w):
    return jnp.zeros((16, w.shape[1]), jnp.float32).at[:w.shape[0]].set(w)


def _pack_cell(p):
    wj = jnp.concatenate([
        _pad16(jnp.concatenate([p[g]['W_jj'] for g in GATES], axis=1)),
        _pad16(jnp.concatenate([p[g]['W_gj'] for g in GATES], axis=1)),
        _pad16(jnp.concatenate([p[g]['W_self_j'] for g in GATES], axis=1)),
        _pad16(jnp.concatenate([p[g]['U_j'] for g in GATES], axis=1)),
    ], axis=0)
    bj = jnp.concatenate([p[g]['b_j'] for g in GATES])[None, :]
    wg = jnp.concatenate([
        _pad16(jnp.concatenate([p[g]['W_jg'] for g in GATES], axis=1)),
        _pad16(jnp.concatenate([p[g]['W_self_g'] for g in GATES], axis=1)),
        _pad16(jnp.concatenate([p[g]['U_g'] for g in GATES], axis=1)),
    ], axis=0)
    bg = jnp.concatenate([p[g]['b_g'] for g in GATES])[None, :]
    return wj, bj, wg, bg


def kernel(x_joint, x_grain, edge_attr_jj, params, edge_index_jj,
           edge_index_jg, edge_index_gj):
    xjp = _pad_rows(x_joint, NJP)
    xgp = _pad_rows(x_grain, NGP)
    s_jj, d_jj = _pad_edges(edge_index_jj, EJJP, NJ)
    s_jg, d_jg = _pad_edges(edge_index_jg, EJGP, NG)
    s_gj, d_gj = _pad_edges(edge_index_gj, EGJP, NJ)
    zj = jnp.zeros((NJP, D), jnp.float32)
    zg = jnp.zeros((NGP, D), jnp.float32)

    seg_jj = _make_segsum(EJJP, NJP)
    seg_gj = _make_segsum(EGJP, NJP)
    seg_jg = _make_segsum(EJGP, NGP)
    cell_j = _make_cell_j(NJP)
    cell_g = _make_cell_g(NGP)

    ones_j = jnp.ones((NJP, D), jnp.float32)
    cjj = seg_jj(ones_j, s_jj, d_jj, zj)
    cgj = seg_gj(ones_j, s_gj, d_gj, zj)
    cjg = seg_jg(ones_j, s_jg, d_jg, zg)

    sjj0 = seg_jj(xjp, s_jj, d_jj, zj)
    sgj0 = seg_gj(xgp, s_gj, d_gj, zj)
    sjg0 = seg_jg(xjp, s_jg, d_jg, zg)

    wj0e, bj0e, wg0e, bg0e = _pack_cell(params['enc'][0])
    wj1e, bj1e, wg1e, bg1e = _pack_cell(params['enc'][1])
    wj0d, bj0d, wg0d, bg0d = _pack_cell(params['dec'][0])
    wj1d, bj1d, _, _ = _pack_cell(params['dec'][1])

    hj0, cj0 = cell_j(sjj0, sgj0, cjj, cgj, xjp, zj, zj, wj0e, bj0e)
    hg0, cg0 = cell_g(sjg0, cjg, xgp, zg, zg, wg0e, bg0e)

    sjj1 = seg_jj(hj0, s_jj, d_jj, zj)
    sgj1 = seg_gj(hg0, s_gj, d_gj, zj)
    sjg1 = seg_jg(hj0, s_jg, d_jg, zg)

    hj1, cj1 = cell_j(sjj1, sgj1, cjj, cgj, hj0, zj, zj, wj1e, bj1e)
    hg1, cg1 = cell_g(sjg1, cjg, hg0, zg, zg, wg1e, bg1e)

    dj0, dcj0 = cell_j(sjj0, sgj0, cjj, cgj, xjp, hj0, cj0, wj0d, bj0d)
    dg0, _ = cell_g(sjg0, cjg, xgp, hg0, cg0, wg0d, bg0d)

    sjj2 = seg_jj(dj0, s_jj, d_jj, zj)
    sgj2 = seg_gj(dg0, s_gj, d_gj, zj)
    dj1, _ = cell_j(sjj2, sgj2, cjj, cgj, dj0, hj1, cj1, wj1d, bj1d)

    w1, w2 = params['lin1_w'][:, 0], params['lin2_w'][:, 0]
    wa = jnp.zeros((D, D), jnp.float32).at[:, 0].set(w1[0:16]) \
        .at[:, 1].set(w2[0:16])
    wb = jnp.zeros((D, D), jnp.float32).at[:, 0].set(w1[16:32]) \
        .at[:, 1].set(w2[16:32])
    tails = jnp.stack([w1[32:34], w2[32:34]], axis=1)
    biases = jnp.stack([params['lin1_b'][0], params['lin2_b'][0]])[None, :]

    ta, tb = _make_project()(dj1, wa, wb)
    gs, gd = _make_pairgather()(ta, tb, s_jj, d_jj)
    ea = jnp.zeros((EJJP, 2), jnp.float32).at[:EJJ].set(edge_attr_jj)
    p = _make_final()(gs, gd, ea, tails, biases)
    return p[:EJJ]


# speedup vs baseline: 11.2626x; 1.2626x over previous
import functools

import jax
import jax.numpy as jnp
from jax import lax
from jax.experimental import pallas as pl
from jax.experimental.pallas import tpu as pltpu
from jax.experimental.pallas import tpu_sc as plsc

NJ, NG = 50000, 25000
OUT = 16
D = 16
EJJ, EJG, EGJ = 800000, 400000, 400000

NC, NS = 2, 16
NW = NC * NS
MB = 128

NJP = 50176
NGP = 25088
EJJP = 819200
EJGP = 425984
EGJP = 425984

GATES = ('i', 'f', 'g', 'o')


def _make_segsum(epad, ndp):
    rt = epad // MB // NW
    zb = ndp // NS

    mesh = plsc.VectorSubcoreMesh(core_axis_name="c", subcore_axis_name="s",
                                  num_cores=NC, num_subcores=NS)

    @functools.partial(
        pl.kernel, mesh=mesh,
        out_type=jax.ShapeDtypeStruct((NC, ndp, D), jnp.float32),
        compiler_params=pltpu.CompilerParams(use_tc_tiling_on_sc=False),
        scratch_types=[
            pltpu.VMEM((rt, MB), jnp.int32),
            pltpu.VMEM((rt, MB), jnp.int32),
            pltpu.VMEM((MB, D), jnp.float32),
            pltpu.VMEM_SHARED((ndp, D), jnp.float32),
            pltpu.SemaphoreType.DMA,
        ])
    def seg(x_hbm, src_hbm, dst_hbm, zeros_hbm, out_hbm,
            sidx, didx, rows, acc, sem):
        cid = lax.axis_index("c")
        sid = lax.axis_index("s")
        wid = cid * NS + sid

        pltpu.sync_copy(zeros_hbm.at[pl.ds(sid * zb, zb)],
                        acc.at[pl.ds(sid * zb, zb)])
        pltpu.sync_copy(src_hbm.at[pl.ds(wid * rt, rt)], sidx)
        pltpu.sync_copy(dst_hbm.at[pl.ds(wid * rt, rt)], didx)
        plsc.subcore_barrier()

        def mb_body(j, carry):
            pltpu.async_copy(x_hbm.at[sidx.at[j]], rows, sem).wait()
            pltpu.sync_copy(rows, acc.at[didx.at[j]], add=True)
            return carry

        lax.fori_loop(0, rt, mb_body, 0)
        plsc.subcore_barrier()

        pltpu.sync_copy(acc.at[pl.ds(sid * zb, zb)],
                        out_hbm.at[cid, pl.ds(sid * zb, zb)])

    return seg


_CE = 2048


def _make_edge_classifier():
    ept = EJJP // NS
    nch = ept // _CE
    mesh = plsc.VectorSubcoreMesh(core_axis_name="c", subcore_axis_name="s",
                                  num_cores=NC, num_subcores=NS)

    @functools.partial(
        pl.kernel, mesh=mesh,
        out_type=(jax.ShapeDtypeStruct((EJJP,), jnp.float32),
                  jax.ShapeDtypeStruct((EJJP,), jnp.float32)),
        compiler_params=pltpu.CompilerParams(use_tc_tiling_on_sc=False,
                                             needs_layout_passes=False),
        scratch_types=[
            pltpu.VMEM((NJP,), jnp.float32),
            pltpu.VMEM((NJP,), jnp.float32),
            pltpu.VMEM((_CE,), jnp.int32),
            pltpu.VMEM((_CE,), jnp.int32),
            pltpu.VMEM((_CE,), jnp.float32),
            pltpu.VMEM((_CE,), jnp.float32),
        ])
    def clf(a1_hbm, b1_hbm, a2_hbm, b2_hbm, src_hbm, dst_hbm,
            eab1_hbm, eab2_hbm, p1_hbm, p2_hbm,
            a_v, b_v, s_v, d_v, e_v, o_v):
        cid = lax.axis_index("c")
        sid = lax.axis_index("s")

        @pl.when(cid == 0)
        def _():
            pltpu.sync_copy(a1_hbm, a_v)
            pltpu.sync_copy(b1_hbm, b_v)

        @pl.when(cid == 1)
        def _():
            pltpu.sync_copy(a2_hbm, a_v)
            pltpu.sync_copy(b2_hbm, b_v)

        def chunk_body(ch, carry):
            gbase = sid * ept + ch * _CE
            pltpu.sync_copy(src_hbm.at[pl.ds(gbase, _CE)], s_v)
            pltpu.sync_copy(dst_hbm.at[pl.ds(gbase, _CE)], d_v)

            @pl.when(cid == 0)
            def _():
                pltpu.sync_copy(eab1_hbm.at[pl.ds(gbase, _CE)], e_v)

            @pl.when(cid == 1)
            def _():
                pltpu.sync_copy(eab2_hbm.at[pl.ds(gbase, _CE)], e_v)

            def vec_body(i, c2):
                sl = pl.ds(i * 16, 16)
                a = plsc.load_gather(a_v, [s_v[sl]])
                b = plsc.load_gather(b_v, [d_v[sl]])
                z = a + b + e_v[sl]
                o_v[sl] = 1.0 / (1.0 + jnp.exp(-z))
                return c2

            lax.fori_loop(0, _CE // 16, vec_body, 0)

            @pl.when(cid == 0)
            def _():
                pltpu.sync_copy(o_v, p1_hbm.at[pl.ds(gbase, _CE)])

            @pl.when(cid == 1)
            def _():
                pltpu.sync_copy(o_v, p2_hbm.at[pl.ds(gbase, _CE)])

            return carry

        lax.fori_loop(0, nch, chunk_body, 0)

    return clf


_BN = 512


def _cell_body_j(sjj, sgj, cjj, cgj, x, h, c, w, b, h2, c2):
    a1 = (sjj[0] + sjj[1]) / jnp.clip(cjj[0] + cjj[1], 1.0, None)
    a2 = (sgj[0] + sgj[1]) / jnp.clip(cgj[0] + cgj[1], 1.0, None)
    wv = w[...]
    z = (jnp.dot(a1, wv[0:16, :], preferred_element_type=jnp.float32,
             precision=lax.Precision.HIGHEST)
         + jnp.dot(a2, wv[16:32, :], preferred_element_type=jnp.float32,
             precision=lax.Precision.HIGHEST)
         + jnp.dot(x[...], wv[32:48, :], preferred_element_type=jnp.float32,
             precision=lax.Precision.HIGHEST)
         + jnp.dot(h[...], wv[48:64, :], preferred_element_type=jnp.float32,
             precision=lax.Precision.HIGHEST)
         + b[...])
    cn = (jax.nn.sigmoid(z[:, 16:32]) * c[...]
          + jax.nn.sigmoid(z[:, 0:16]) * jnp.tanh(z[:, 32:48]))
    h2[...] = jax.nn.sigmoid(z[:, 48:64]) * jnp.tanh(cn)
    c2[...] = cn


def _cell_body_g(sjg, cjg, x, h, c, w, b, h2, c2):
    a1 = (sjg[0] + sjg[1]) / jnp.clip(cjg[0] + cjg[1], 1.0, None)
    wv = w[...]
    z = (jnp.dot(a1, wv[0:16, :], preferred_element_type=jnp.float32,
             precision=lax.Precision.HIGHEST)
         + jnp.dot(x[...], wv[16:32, :], preferred_element_type=jnp.float32,
             precision=lax.Precision.HIGHEST)
         + jnp.dot(h[...], wv[32:48, :], preferred_element_type=jnp.float32,
             precision=lax.Precision.HIGHEST)
         + b[...])
    cn = (jax.nn.sigmoid(z[:, 16:32]) * c[...]
          + jax.nn.sigmoid(z[:, 0:16]) * jnp.tanh(z[:, 32:48]))
    h2[...] = jax.nn.sigmoid(z[:, 48:64]) * jnp.tanh(cn)
    c2[...] = cn


def _make_cell_j(ndp):
    grid = (ndp // _BN,)
    part = lambda i: (0, i, 0)
    row = lambda i: (i, 0)
    return pl.pallas_call(
        _cell_body_j,
        grid=grid,
        in_specs=[
            pl.BlockSpec((NC, _BN, D), part),
            pl.BlockSpec((NC, _BN, D), part),
            pl.BlockSpec((NC, _BN, D), part),
            pl.BlockSpec((NC, _BN, D), part),
            pl.BlockSpec((_BN, D), row),
            pl.BlockSpec((_BN, D), row),
            pl.BlockSpec((_BN, D), row),
            pl.BlockSpec((64, 64), lambda i: (0, 0)),
            pl.BlockSpec((1, 64), lambda i: (0, 0)),
        ],
        out_specs=[pl.BlockSpec((_BN, D), row), pl.BlockSpec((_BN, D), row)],
        out_shape=[jax.ShapeDtypeStruct((ndp, D), jnp.float32),
                   jax.ShapeDtypeStruct((ndp, D), jnp.float32)],
    )


def _make_cell_g(ndp):
    grid = (ndp // _BN,)
    part = lambda i: (0, i, 0)
    row = lambda i: (i, 0)
    return pl.pallas_call(
        _cell_body_g,
        grid=grid,
        in_specs=[
            pl.BlockSpec((NC, _BN, D), part),
            pl.BlockSpec((NC, _BN, D), part),
            pl.BlockSpec((_BN, D), row),
            pl.BlockSpec((_BN, D), row),
            pl.BlockSpec((_BN, D), row),
            pl.BlockSpec((48, 64), lambda i: (0, 0)),
            pl.BlockSpec((1, 64), lambda i: (0, 0)),
        ],
        out_specs=[pl.BlockSpec((_BN, D), row), pl.BlockSpec((_BN, D), row)],
        out_shape=[jax.ShapeDtypeStruct((ndp, D), jnp.float32),
                   jax.ShapeDtypeStruct((ndp, D), jnp.float32)],
    )


def _project_body(h, w4, a1, b1, a2, b2):
    t = jnp.dot(h[...], w4[...], preferred_element_type=jnp.float32,
                precision=lax.Precision.HIGHEST)
    a1[...] = t[:, 0]
    b1[...] = t[:, 1]
    a2[...] = t[:, 2]
    b2[...] = t[:, 3]


def _eab_body(ea, wt, b, e1, e2):
    t = (jnp.dot(ea[...], wt[...], preferred_element_type=jnp.float32,
                 precision=lax.Precision.HIGHEST) + b[...])
    e1[...] = t[:, 0]
    e2[...] = t[:, 1]


def _make_eab():
    be = 4096
    grid = (EJJP // be,)
    col = pl.BlockSpec((be,), lambda i: (i,))
    return pl.pallas_call(
        _eab_body,
        grid=grid,
        in_specs=[pl.BlockSpec((be, 2), lambda i: (i, 0)),
                  pl.BlockSpec((2, 2), lambda i: (0, 0)),
                  pl.BlockSpec((1, 2), lambda i: (0, 0))],
        out_specs=[col, col],
        out_shape=[jax.ShapeDtypeStruct((EJJP,), jnp.float32)] * 2,
    )


def _make_project():
    grid = (NJP // _BN,)
    col = pl.BlockSpec((_BN,), lambda i: (i,))
    return pl.pallas_call(
        _project_body,
        grid=grid,
        in_specs=[pl.BlockSpec((_BN, D), lambda i: (i, 0)),
                  pl.BlockSpec((D, 4), lambda i: (0, 0))],
        out_specs=[col, col, col, col],
        out_shape=[jax.ShapeDtypeStruct((NJP,), jnp.float32)] * 4,
    )


def _pad_rows(x, n, d=D):
    return jnp.zeros((n, d), jnp.float32).at[:x.shape[0], :x.shape[1]].set(x)


def _pad_edges(ei, epad, dst_pad):
    e = ei.shape[1]
    src = jnp.full((epad,), 0, jnp.int32).at[:e].set(ei[0].astype(jnp.int32))
    dst = jnp.full((epad,), dst_pad, jnp.int32).at[:e].set(
        ei[1].astype(jnp.int32))
    return src.reshape(epad // MB, MB), dst.reshape(epad // MB, MB)


def _pad16(w):
    return jnp.zeros((16, w.shape[1]), jnp.float32).at[:w.shape[0]].set(w)


def _pack_cell(p):
    wj = jnp.concatenate([
        _pad16(jnp.concatenate([p[g]['W_jj'] for g in GATES], axis=1)),
        _pad16(jnp.concatenate([p[g]['W_gj'] for g in GATES], axis=1)),
        _pad16(jnp.concatenate([p[g]['W_self_j'] for g in GATES], axis=1)),
        _pad16(jnp.concatenate([p[g]['U_j'] for g in GATES], axis=1)),
    ], axis=0)
    bj = jnp.concatenate([p[g]['b_j'] for g in GATES])[None, :]
    wg = jnp.concatenate([
        _pad16(jnp.concatenate([p[g]['W_jg'] for g in GATES], axis=1)),
        _pad16(jnp.concatenate([p[g]['W_self_g'] for g in GATES], axis=1)),
        _pad16(jnp.concatenate([p[g]['U_g'] for g in GATES], axis=1)),
    ], axis=0)
    bg = jnp.concatenate([p[g]['b_g'] for g in GATES])[None, :]
    return wj, bj, wg, bg


def kernel(x_joint, x_grain, edge_attr_jj, params, edge_index_jj,
           edge_index_jg, edge_index_gj):
    xjp = _pad_rows(x_joint, NJP)
    xgp = _pad_rows(x_grain, NGP)
    s_jj, d_jj = _pad_edges(edge_index_jj, EJJP, NJ)
    s_jg, d_jg = _pad_edges(edge_index_jg, EJGP, NG)
    s_gj, d_gj = _pad_edges(edge_index_gj, EGJP, NJ)
    zj = jnp.zeros((NJP, D), jnp.float32)
    zg = jnp.zeros((NGP, D), jnp.float32)

    seg_jj = _make_segsum(EJJP, NJP)
    seg_gj = _make_segsum(EGJP, NJP)
    seg_jg = _make_segsum(EJGP, NGP)
    cell_j = _make_cell_j(NJP)
    cell_g = _make_cell_g(NGP)

    ones_j = jnp.ones((NJP, D), jnp.float32)
    cjj = seg_jj(ones_j, s_jj, d_jj, zj)
    cgj = seg_gj(ones_j, s_gj, d_gj, zj)
    cjg = seg_jg(ones_j, s_jg, d_jg, zg)

    sjj0 = seg_jj(xjp, s_jj, d_jj, zj)
    sgj0 = seg_gj(xgp, s_gj, d_gj, zj)
    sjg0 = seg_jg(xjp, s_jg, d_jg, zg)

    wj0e, bj0e, wg0e, bg0e = _pack_cell(params['enc'][0])
    wj1e, bj1e, wg1e, bg1e = _pack_cell(params['enc'][1])
    wj0d, bj0d, wg0d, bg0d = _pack_cell(params['dec'][0])
    wj1d, bj1d, _, _ = _pack_cell(params['dec'][1])

    hj0, cj0 = cell_j(sjj0, sgj0, cjj, cgj, xjp, zj, zj, wj0e, bj0e)
    hg0, cg0 = cell_g(sjg0, cjg, xgp, zg, zg, wg0e, bg0e)

    sjj1 = seg_jj(hj0, s_jj, d_jj, zj)
    sgj1 = seg_gj(hg0, s_gj, d_gj, zj)
    sjg1 = seg_jg(hj0, s_jg, d_jg, zg)

    hj1, cj1 = cell_j(sjj1, sgj1, cjj, cgj, hj0, zj, zj, wj1e, bj1e)
    hg1, cg1 = cell_g(sjg1, cjg, hg0, zg, zg, wg1e, bg1e)

    dj0, dcj0 = cell_j(sjj0, sgj0, cjj, cgj, xjp, hj0, cj0, wj0d, bj0d)
    dg0, _ = cell_g(sjg0, cjg, xgp, hg0, cg0, wg0d, bg0d)

    sjj2 = seg_jj(dj0, s_jj, d_jj, zj)
    sgj2 = seg_gj(dg0, s_gj, d_gj, zj)
    dj1, _ = cell_j(sjj2, sgj2, cjj, cgj, dj0, hj1, cj1, wj1d, bj1d)

    w1, w2 = params['lin1_w'][:, 0], params['lin2_w'][:, 0]
    w4 = jnp.stack([w1[0:16], w1[16:32], w2[0:16], w2[16:32]], axis=1)
    tails = jnp.stack([w1[32:34], w2[32:34]], axis=1)
    biases = jnp.stack([params['lin1_b'][0], params['lin2_b'][0]])[None, :]

    a1, b1, a2, b2 = _make_project()(dj1, w4)
    ea = jnp.zeros((EJJP, 2), jnp.float32).at[:EJJ].set(edge_attr_jj)
    eab1, eab2 = _make_eab()(ea, tails, biases)
    p1, p2 = _make_edge_classifier()(
        a1, b1, a2, b2, s_jj.reshape(-1), d_jj.reshape(-1), eab1, eab2)
    return jnp.stack([p1[:EJJ], p2[:EJJ]], axis=1)


# speedup vs baseline: 11.9797x; 1.0637x over previous
import functools

import jax
import jax.numpy as jnp
from jax import lax
from jax.experimental import pallas as pl
from jax.experimental.pallas import tpu as pltpu
from jax.experimental.pallas import tpu_sc as plsc

NJ, NG = 50000, 25000
OUT = 16
D = 16
EJJ, EJG, EGJ = 800000, 400000, 400000

NC, NS = 2, 16
NW = NC * NS
MB = 128

NJP = 50176
NGP = 25088
EJJP = 819200
EJGP = 425984
EGJP = 425984

GATES = ('i', 'f', 'g', 'o')


def _make_segsum(epad, ndp):
    rt = epad // MB // NW
    zb = ndp // NS
    nbuf = 8
    assert rt % nbuf == 0

    mesh = plsc.VectorSubcoreMesh(core_axis_name="c", subcore_axis_name="s",
                                  num_cores=NC, num_subcores=NS)

    @functools.partial(
        pl.kernel, mesh=mesh,
        out_type=jax.ShapeDtypeStruct((NC, ndp, D), jnp.float32),
        compiler_params=pltpu.CompilerParams(use_tc_tiling_on_sc=False),
        scratch_types=[
            pltpu.VMEM((rt, MB), jnp.int32),
            pltpu.VMEM((rt, MB), jnp.int32),
            pltpu.VMEM((nbuf, MB, D), jnp.float32),
            pltpu.VMEM_SHARED((ndp, D), jnp.float32),
        ] + [pltpu.SemaphoreType.DMA] * (2 * nbuf))
    def seg(x_hbm, src_hbm, dst_hbm, zeros_hbm, out_hbm,
            sidx, didx, rows, acc, *sems):
        semg, sems_ = sems[:nbuf], sems[nbuf:]
        cid = lax.axis_index("c")
        sid = lax.axis_index("s")
        wid = cid * NS + sid

        pltpu.sync_copy(zeros_hbm.at[pl.ds(sid * zb, zb)],
                        acc.at[pl.ds(sid * zb, zb)])
        pltpu.sync_copy(src_hbm.at[pl.ds(wid * rt, rt)], sidx)
        pltpu.sync_copy(dst_hbm.at[pl.ds(wid * rt, rt)], didx)
        plsc.subcore_barrier()

        for b in range(nbuf):
            pltpu.async_copy(x_hbm.at[sidx.at[b]], rows.at[b], semg[b])

        def round_body(r, carry):
            j0 = r * nbuf
            descs = []
            for b in range(nbuf):
                pltpu.make_async_copy(x_hbm.at[sidx.at[j0 + b]], rows.at[b],
                                      semg[b]).wait()
                descs.append(pltpu.async_copy(
                    rows.at[b], acc.at[didx.at[j0 + b]], sems_[b], add=True))
            for b in range(nbuf):
                descs[b].wait()
                jn = j0 + b + nbuf

                @pl.when(jn < rt)
                def _(b=b, jn=jn):
                    pltpu.async_copy(x_hbm.at[sidx.at[jn]], rows.at[b],
                                     semg[b])
            return carry

        lax.fori_loop(0, rt // nbuf, round_body, 0)
        plsc.subcore_barrier()

        pltpu.sync_copy(acc.at[pl.ds(sid * zb, zb)],
                        out_hbm.at[cid, pl.ds(sid * zb, zb)])

    return seg


_CE = 2048


def _make_edge_classifier():
    ept = EJJP // NS
    nch = ept // _CE
    mesh = plsc.VectorSubcoreMesh(core_axis_name="c", subcore_axis_name="s",
                                  num_cores=NC, num_subcores=NS)

    @functools.partial(
        pl.kernel, mesh=mesh,
        out_type=(jax.ShapeDtypeStruct((EJJP,), jnp.float32),
                  jax.ShapeDtypeStruct((EJJP,), jnp.float32)),
        compiler_params=pltpu.CompilerParams(use_tc_tiling_on_sc=False,
                                             needs_layout_passes=False),
        scratch_types=[
            pltpu.VMEM((NJP,), jnp.float32),
            pltpu.VMEM((NJP,), jnp.float32),
            pltpu.VMEM((_CE,), jnp.int32),
            pltpu.VMEM((_CE,), jnp.int32),
            pltpu.VMEM((_CE,), jnp.float32),
            pltpu.VMEM((_CE,), jnp.float32),
        ])
    def clf(a1_hbm, b1_hbm, a2_hbm, b2_hbm, src_hbm, dst_hbm,
            eab1_hbm, eab2_hbm, p1_hbm, p2_hbm,
            a_v, b_v, s_v, d_v, e_v, o_v):
        cid = lax.axis_index("c")
        sid = lax.axis_index("s")

        @pl.when(cid == 0)
        def _():
            pltpu.sync_copy(a1_hbm, a_v)
            pltpu.sync_copy(b1_hbm, b_v)

        @pl.when(cid == 1)
        def _():
            pltpu.sync_copy(a2_hbm, a_v)
            pltpu.sync_copy(b2_hbm, b_v)

        def chunk_body(ch, carry):
            gbase = sid * ept + ch * _CE
            pltpu.sync_copy(src_hbm.at[pl.ds(gbase, _CE)], s_v)
            pltpu.sync_copy(dst_hbm.at[pl.ds(gbase, _CE)], d_v)

            @pl.when(cid == 0)
            def _():
                pltpu.sync_copy(eab1_hbm.at[pl.ds(gbase, _CE)], e_v)

            @pl.when(cid == 1)
            def _():
                pltpu.sync_copy(eab2_hbm.at[pl.ds(gbase, _CE)], e_v)

            def vec_body(i, c2):
                sl = pl.ds(i * 16, 16)
                a = plsc.load_gather(a_v, [s_v[sl]])
                b = plsc.load_gather(b_v, [d_v[sl]])
                z = a + b + e_v[sl]
                o_v[sl] = 1.0 / (1.0 + jnp.exp(-z))
                return c2

            lax.fori_loop(0, _CE // 16, vec_body, 0)

            @pl.when(cid == 0)
            def _():
                pltpu.sync_copy(o_v, p1_hbm.at[pl.ds(gbase, _CE)])

            @pl.when(cid == 1)
            def _():
                pltpu.sync_copy(o_v, p2_hbm.at[pl.ds(gbase, _CE)])

            return carry

        lax.fori_loop(0, nch, chunk_body, 0)

    return clf


_BN = 512


def _cell_body_j(sjj, sgj, cjj, cgj, x, h, c, w, b, h2, c2):
    a1 = (sjj[0] + sjj[1]) / jnp.clip(cjj[0] + cjj[1], 1.0, None)
    a2 = (sgj[0] + sgj[1]) / jnp.clip(cgj[0] + cgj[1], 1.0, None)
    wv = w[...]
    z = (jnp.dot(a1, wv[0:16, :], preferred_element_type=jnp.float32,
             precision=lax.Precision.HIGHEST)
         + jnp.dot(a2, wv[16:32, :], preferred_element_type=jnp.float32,
             precision=lax.Precision.HIGHEST)
         + jnp.dot(x[...], wv[32:48, :], preferred_element_type=jnp.float32,
             precision=lax.Precision.HIGHEST)
         + jnp.dot(h[...], wv[48:64, :], preferred_element_type=jnp.float32,
             precision=lax.Precision.HIGHEST)
         + b[...])
    cn = (jax.nn.sigmoid(z[:, 16:32]) * c[...]
          + jax.nn.sigmoid(z[:, 0:16]) * jnp.tanh(z[:, 32:48]))
    h2[...] = jax.nn.sigmoid(z[:, 48:64]) * jnp.tanh(cn)
    c2[...] = cn


def _cell_body_g(sjg, cjg, x, h, c, w, b, h2, c2):
    a1 = (sjg[0] + sjg[1]) / jnp.clip(cjg[0] + cjg[1], 1.0, None)
    wv = w[...]
    z = (jnp.dot(a1, wv[0:16, :], preferred_element_type=jnp.float32,
             precision=lax.Precision.HIGHEST)
         + jnp.dot(x[...], wv[16:32, :], preferred_element_type=jnp.float32,
             precision=lax.Precision.HIGHEST)
         + jnp.dot(h[...], wv[32:48, :], preferred_element_type=jnp.float32,
             precision=lax.Precision.HIGHEST)
         + b[...])
    cn = (jax.nn.sigmoid(z[:, 16:32]) * c[...]
          + jax.nn.sigmoid(z[:, 0:16]) * jnp.tanh(z[:, 32:48]))
    h2[...] = jax.nn.sigmoid(z[:, 48:64]) * jnp.tanh(cn)
    c2[...] = cn


def _make_cell_j(ndp):
    grid = (ndp // _BN,)
    part = lambda i: (0, i, 0)
    row = lambda i: (i, 0)
    return pl.pallas_call(
        _cell_body_j,
        grid=grid,
        in_specs=[
            pl.BlockSpec((NC, _BN, D), part),
            pl.BlockSpec((NC, _BN, D), part),
            pl.BlockSpec((NC, _BN, D), part),
            pl.BlockSpec((NC, _BN, D), part),
            pl.BlockSpec((_BN, D), row),
            pl.BlockSpec((_BN, D), row),
            pl.BlockSpec((_BN, D), row),
            pl.BlockSpec((64, 64), lambda i: (0, 0)),
            pl.BlockSpec((1, 64), lambda i: (0, 0)),
        ],
        out_specs=[pl.BlockSpec((_BN, D), row), pl.BlockSpec((_BN, D), row)],
        out_shape=[jax.ShapeDtypeStruct((ndp, D), jnp.float32),
                   jax.ShapeDtypeStruct((ndp, D), jnp.float32)],
    )


def _make_cell_g(ndp):
    grid = (ndp // _BN,)
    part = lambda i: (0, i, 0)
    row = lambda i: (i, 0)
    return pl.pallas_call(
        _cell_body_g,
        grid=grid,
        in_specs=[
            pl.BlockSpec((NC, _BN, D), part),
            pl.BlockSpec((NC, _BN, D), part),
            pl.BlockSpec((_BN, D), row),
            pl.BlockSpec((_BN, D), row),
            pl.BlockSpec((_BN, D), row),
            pl.BlockSpec((48, 64), lambda i: (0, 0)),
            pl.BlockSpec((1, 64), lambda i: (0, 0)),
        ],
        out_specs=[pl.BlockSpec((_BN, D), row), pl.BlockSpec((_BN, D), row)],
        out_shape=[jax.ShapeDtypeStruct((ndp, D), jnp.float32),
                   jax.ShapeDtypeStruct((ndp, D), jnp.float32)],
    )


def _project_body(h, w4, a1, b1, a2, b2):
    t = jnp.dot(h[...], w4[...], preferred_element_type=jnp.float32,
                precision=lax.Precision.HIGHEST)
    a1[...] = t[:, 0]
    b1[...] = t[:, 1]
    a2[...] = t[:, 2]
    b2[...] = t[:, 3]


def _eab_body(ea, wt, b, e1, e2):
    t = (jnp.dot(ea[...], wt[...], preferred_element_type=jnp.float32,
                 precision=lax.Precision.HIGHEST) + b[...])
    e1[...] = t[:, 0]
    e2[...] = t[:, 1]


def _make_eab():
    be = 4096
    grid = (EJJP // be,)
    col = pl.BlockSpec((be,), lambda i: (i,))
    return pl.pallas_call(
        _eab_body,
        grid=grid,
        in_specs=[pl.BlockSpec((be, 2), lambda i: (i, 0)),
                  pl.BlockSpec((2, 2), lambda i: (0, 0)),
                  pl.BlockSpec((1, 2), lambda i: (0, 0))],
        out_specs=[col, col],
        out_shape=[jax.ShapeDtypeStruct((EJJP,), jnp.float32)] * 2,
    )


def _make_project():
    grid = (NJP // _BN,)
    col = pl.BlockSpec((_BN,), lambda i: (i,))
    return pl.pallas_call(
        _project_body,
        grid=grid,
        in_specs=[pl.BlockSpec((_BN, D), lambda i: (i, 0)),
                  pl.BlockSpec((D, 4), lambda i: (0, 0))],
        out_specs=[col, col, col, col],
        out_shape=[jax.ShapeDtypeStruct((NJP,), jnp.float32)] * 4,
    )


def _pad_rows(x, n, d=D):
    return jnp.zeros((n, d), jnp.float32).at[:x.shape[0], :x.shape[1]].set(x)


def _pad_edges(ei, epad, dst_pad):
    e = ei.shape[1]
    src = jnp.full((epad,), 0, jnp.int32).at[:e].set(ei[0].astype(jnp.int32))
    dst = jnp.full((epad,), dst_pad, jnp.int32).at[:e].set(
        ei[1].astype(jnp.int32))
    return src.reshape(epad // MB, MB), dst.reshape(epad // MB, MB)


def _pad16(w):
    return jnp.zeros((16, w.shape[1]), jnp.float32).at[:w.shape[0]].set(w)


def _pack_cell(p):
    wj = jnp.concatenate([
        _pad16(jnp.concatenate([p[g]['W_jj'] for g in GATES], axis=1)),
        _pad16(jnp.concatenate([p[g]['W_gj'] for g in GATES], axis=1)),
        _pad16(jnp.concatenate([p[g]['W_self_j'] for g in GATES], axis=1)),
        _pad16(jnp.concatenate([p[g]['U_j'] for g in GATES], axis=1)),
    ], axis=0)
    bj = jnp.concatenate([p[g]['b_j'] for g in GATES])[None, :]
    wg = jnp.concatenate([
        _pad16(jnp.concatenate([p[g]['W_jg'] for g in GATES], axis=1)),
        _pad16(jnp.concatenate([p[g]['W_self_g'] for g in GATES], axis=1)),
        _pad16(jnp.concatenate([p[g]['U_g'] for g in GATES], axis=1)),
    ], axis=0)
    bg = jnp.concatenate([p[g]['b_g'] for g in GATES])[None, :]
    return wj, bj, wg, bg


def kernel(x_joint, x_grain, edge_attr_jj, params, edge_index_jj,
           edge_index_jg, edge_index_gj):
    xjp = _pad_rows(x_joint, NJP)
    xgp = _pad_rows(x_grain, NGP)
    s_jj, d_jj = _pad_edges(edge_index_jj, EJJP, NJ)
    s_jg, d_jg = _pad_edges(edge_index_jg, EJGP, NG)
    s_gj, d_gj = _pad_edges(edge_index_gj, EGJP, NJ)
    zj = jnp.zeros((NJP, D), jnp.float32)
    zg = jnp.zeros((NGP, D), jnp.float32)

    seg_jj = _make_segsum(EJJP, NJP)
    seg_gj = _make_segsum(EGJP, NJP)
    seg_jg = _make_segsum(EJGP, NGP)
    cell_j = _make_cell_j(NJP)
    cell_g = _make_cell_g(NGP)

    ones_j = jnp.ones((NJP, D), jnp.float32)
    cjj = seg_jj(ones_j, s_jj, d_jj, zj)
    cgj = seg_gj(ones_j, s_gj, d_gj, zj)
    cjg = seg_jg(ones_j, s_jg, d_jg, zg)

    sjj0 = seg_jj(xjp, s_jj, d_jj, zj)
    sgj0 = seg_gj(xgp, s_gj, d_gj, zj)
    sjg0 = seg_jg(xjp, s_jg, d_jg, zg)

    wj0e, bj0e, wg0e, bg0e = _pack_cell(params['enc'][0])
    wj1e, bj1e, wg1e, bg1e = _pack_cell(params['enc'][1])
    wj0d, bj0d, wg0d, bg0d = _pack_cell(params['dec'][0])
    wj1d, bj1d, _, _ = _pack_cell(params['dec'][1])

    hj0, cj0 = cell_j(sjj0, sgj0, cjj, cgj, xjp, zj, zj, wj0e, bj0e)
    hg0, cg0 = cell_g(sjg0, cjg, xgp, zg, zg, wg0e, bg0e)

    sjj1 = seg_jj(hj0, s_jj, d_jj, zj)
    sgj1 = seg_gj(hg0, s_gj, d_gj, zj)
    sjg1 = seg_jg(hj0, s_jg, d_jg, zg)

    hj1, cj1 = cell_j(sjj1, sgj1, cjj, cgj, hj0, zj, zj, wj1e, bj1e)
    hg1, cg1 = cell_g(sjg1, cjg, hg0, zg, zg, wg1e, bg1e)

    dj0, dcj0 = cell_j(sjj0, sgj0, cjj, cgj, xjp, hj0, cj0, wj0d, bj0d)
    dg0, _ = cell_g(sjg0, cjg, xgp, hg0, cg0, wg0d, bg0d)

    sjj2 = seg_jj(dj0, s_jj, d_jj, zj)
    sgj2 = seg_gj(dg0, s_gj, d_gj, zj)
    dj1, _ = cell_j(sjj2, sgj2, cjj, cgj, dj0, hj1, cj1, wj1d, bj1d)

    w1, w2 = params['lin1_w'][:, 0], params['lin2_w'][:, 0]
    w4 = jnp.stack([w1[0:16], w1[16:32], w2[0:16], w2[16:32]], axis=1)
    tails = jnp.stack([w1[32:34], w2[32:34]], axis=1)
    biases = jnp.stack([params['lin1_b'][0], params['lin2_b'][0]])[None, :]

    a1, b1, a2, b2 = _make_project()(dj1, w4)
    ea = jnp.zeros((EJJP, 2), jnp.float32).at[:EJJ].set(edge_attr_jj)
    eab1, eab2 = _make_eab()(ea, tails, biases)
    p1, p2 = _make_edge_classifier()(
        a1, b1, a2, b2, s_jj.reshape(-1), d_jj.reshape(-1), eab1, eab2)
    return jnp.stack([p1[:EJJ], p2[:EJJ]], axis=1)
